# baseline (speedup 1.0000x reference)
import functools

import jax
import jax.numpy as jnp
from jax import lax
from jax.experimental import pallas as pl
from jax.experimental.pallas import tpu as pltpu
from jax.experimental.pallas import tpu_sc as plsc

B, L, D = 16384, 50, 64
NC, NS = 2, 16
NW = NC * NS
BPW = B // NW
IPC = 2
CI = IPC * L
NCHUNK = BPW // IPC
NBUF = 3
LANES = 16
QS = D // LANES


def _body(table_hbm, idx_hbm, out_hbm, idx_v, out_v,
          buf0, buf1, buf2, sem0, sem1, sem2):
    c = lax.axis_index("c")
    s = lax.axis_index("s")
    wid = s * NC + c

    pltpu.sync_copy(idx_hbm.at[wid], idx_v)

    bufs = (buf0, buf1, buf2)
    sems = (sem0, sem1, sem2)

    def gather_start(chunk, b):
        pltpu.make_async_copy(
            table_hbm.at[idx_v.at[chunk]], bufs[b], sems[b]
        ).start()

    def gather_wait(chunk, b):
        pltpu.make_async_copy(
            table_hbm.at[idx_v.at[chunk]], bufs[b], sems[b]
        ).wait()

    for b in range(2):
        gather_start(b, b)

    def consume(chunk, b):
        buf = bufs[b]
        for i in range(IPC):
            base = i * L
            accs = [buf[base, pl.ds(q * LANES, LANES)] for q in range(QS)]
            for r in range(1, L):
                for q in range(QS):
                    accs[q] = accs[q] + buf[base + r, pl.ds(q * LANES, LANES)]
            row = chunk * IPC + i
            for q in range(QS):
                out_v[row, pl.ds(q * LANES, LANES)] = accs[q]

    def loop_body(g, carry):
        for b in range(NBUF):
            chunk = g * NBUF + b
            gather_wait(chunk, b)
            @pl.when(chunk + 2 < NCHUNK)
            def _():
                gather_start(chunk + 2, (b + 2) % NBUF)
            consume(chunk, b)
        return carry

    lax.fori_loop(0, (NCHUNK - 1) // NBUF, loop_body, 0)
    gather_wait(NCHUNK - 1, (NCHUNK - 1) % NBUF)
    consume(NCHUNK - 1, (NCHUNK - 1) % NBUF)

    pltpu.sync_copy(out_v, out_hbm.at[pl.ds(wid * BPW, BPW)])


_sc_call = functools.partial(
    pl.kernel,
    out_type=jax.ShapeDtypeStruct((B, D), jnp.float32),
    mesh=plsc.VectorSubcoreMesh(
        core_axis_name="c", subcore_axis_name="s",
        num_cores=NC, num_subcores=NS,
    ),
    scratch_types=[
        pltpu.VMEM((NCHUNK, CI), jnp.int32),
        pltpu.VMEM((BPW, D), jnp.float32),
        pltpu.VMEM((CI, D), jnp.float32),
        pltpu.VMEM((CI, D), jnp.float32),
        pltpu.VMEM((CI, D), jnp.float32),
        pltpu.SemaphoreType.DMA,
        pltpu.SemaphoreType.DMA,
        pltpu.SemaphoreType.DMA,
    ],
    compiler_params=pltpu.CompilerParams(use_tc_tiling_on_sc=False),
)(_body)


@jax.jit
def kernel(tag_ids, table):
    idx = tag_ids.reshape(NW, NCHUNK, CI)
    return _sc_call(table, idx)


# speedup vs baseline: 1.1371x; 1.1371x over previous
import functools

import jax
import jax.numpy as jnp
from jax import lax
from jax.experimental import pallas as pl
from jax.experimental.pallas import tpu as pltpu
from jax.experimental.pallas import tpu_sc as plsc

B, L, D = 16384, 50, 64
NC, NS = 2, 16
NW = NC * NS
BPW = B // NW
IG = 128
G = BPW // IG
LANES = 16
QS = D // LANES


def _body(table_hbm, idx_hbm, out_hbm, idx_v, buf0, buf1, buf2, buf3,
          sem0, sem1, sem2, sem3):
    c = lax.axis_index("c")
    s = lax.axis_index("s")
    wid = s * NC + c

    pltpu.sync_copy(idx_hbm.at[wid], idx_v)

    bufs = (buf0, buf1, buf2, buf3)
    sems = (sem0, sem1, sem2, sem3)

    def fire(g, r, b, add):
        pltpu.async_copy(table_hbm.at[idx_v.at[g, r]], bufs[b], sems[b],
                         add=add)

    def wait_one(b):
        pltpu.make_async_copy(
            table_hbm.at[idx_v.at[0, 0]], bufs[b], sems[b]
        ).wait()

    for g in range(G):
        fire(g, 0, g, False)

    def rloop(r, carry):
        for g in range(G):
            wait_one(g)
            fire(g, r, g, True)
        return carry

    lax.fori_loop(1, L, rloop, 0)

    for g in range(G):
        wait_one(g)
        pltpu.sync_copy(bufs[g], out_hbm.at[pl.ds(wid * BPW + g * IG, IG)])


_sc_call = functools.partial(
    pl.kernel,
    out_type=jax.ShapeDtypeStruct((B, D), jnp.float32),
    mesh=plsc.VectorSubcoreMesh(
        core_axis_name="c", subcore_axis_name="s",
        num_cores=NC, num_subcores=NS,
    ),
    scratch_types=[
        pltpu.VMEM((G, L, IG), jnp.int32),
        pltpu.VMEM((IG, D), jnp.float32),
        pltpu.VMEM((IG, D), jnp.float32),
        pltpu.VMEM((IG, D), jnp.float32),
        pltpu.VMEM((IG, D), jnp.float32),
        pltpu.SemaphoreType.DMA,
        pltpu.SemaphoreType.DMA,
        pltpu.SemaphoreType.DMA,
        pltpu.SemaphoreType.DMA,
    ],
    compiler_params=pltpu.CompilerParams(use_tc_tiling_on_sc=False),
)(_body)


@jax.jit
def kernel(tag_ids, table):
    idx = tag_ids.reshape(NW, G, IG, L).transpose(0, 1, 3, 2)
    return _sc_call(table, idx)
